# Initial kernel scaffold; baseline (speedup 1.0000x reference)
import jax
import jax.numpy as jnp
from jax.experimental import pallas as pl


def kernel(x, edge_index, W_self, W_neigh, b_neigh, W1, b1, W2, b2):
    raise NotImplementedError("write your pallas kernel here")


# speedup vs baseline: 4.8568x; 4.8568x over previous
import jax
import jax.numpy as jnp
from jax import lax
from jax.experimental import pallas as pl
from jax.experimental.pallas import tpu as pltpu
from jax.experimental.pallas import tpu_sc as plsc

N = 10000
E = 320000
D = 128

NC = 2
NS = 16
NW = NC * NS
EPW = E // NW
CH = 80
NCH = EPW // CH
RPT = 624
REM = N - NS * RPT
ZR = 48


def _sc_body(x_hbm, src_hbm, dst_hbm, sum_out, cnt_out,
             acc, idx_s, idx_d, rows_v, sem):
    c = lax.axis_index("c")
    s = lax.axis_index("s")
    wid = s * NC + c
    rbase = s * RPT
    ebase = wid * EPW

    def _fill_rows(nrows, vec):
        def _f(i, _):
            for k in range(D // 16):
                rows_v[i, pl.ds(16 * k, 16)] = vec
            return 0
        lax.fori_loop(0, nrows, _f, 0)

    def _zero_acc():
        for m in range(RPT // ZR):
            pltpu.sync_copy(rows_v.at[pl.ds(0, ZR)],
                            acc.at[pl.ds(rbase + m * ZR, ZR)])

        @pl.when(s == NS - 1)
        def _zero_rem():
            pltpu.sync_copy(rows_v.at[pl.ds(0, REM)],
                            acc.at[pl.ds(NS * RPT, REM)])

    def _writeback(out_hbm):
        def _wb(base, rows):
            pltpu.sync_copy(acc.at[pl.ds(base, rows)], rows_v.at[pl.ds(0, rows)])
            pltpu.sync_copy(rows_v.at[pl.ds(0, rows)],
                            out_hbm.at[c, pl.ds(base, rows)])

        for m in range(RPT // ZR):
            _wb(rbase + m * ZR, ZR)

        @pl.when(s == NS - 1)
        def _wb_rem():
            _wb(NS * RPT, REM)

    zero16 = jnp.zeros((16,), jnp.float32)
    one16 = jnp.ones((16,), jnp.float32)

    _fill_rows(ZR, zero16)
    _zero_acc()
    plsc.subcore_barrier()

    def _chunk1(j, _):
        off = pl.multiple_of(ebase + j * CH, 8)
        pltpu.sync_copy(src_hbm.at[pl.ds(off, CH)], idx_s)
        pltpu.sync_copy(dst_hbm.at[pl.ds(off, CH)], idx_d)
        pltpu.async_copy(x_hbm.at[idx_s], rows_v, sem).wait()
        pltpu.sync_copy(rows_v, acc.at[idx_d], add=True)
        return 0
    lax.fori_loop(0, NCH, _chunk1, 0)
    plsc.subcore_barrier()

    _writeback(sum_out)
    plsc.subcore_barrier()

    _fill_rows(ZR, zero16)
    _zero_acc()
    plsc.subcore_barrier()
    _fill_rows(CH, one16)

    def _chunk2(j, _):
        off = pl.multiple_of(ebase + j * CH, 8)
        pltpu.sync_copy(dst_hbm.at[pl.ds(off, CH)], idx_d)
        pltpu.sync_copy(rows_v, acc.at[idx_d], add=True)
        return 0
    lax.fori_loop(0, NCH, _chunk2, 0)
    plsc.subcore_barrier()

    _writeback(cnt_out)


@jax.jit
def _sc_aggregate(x, src, dst):
    mesh = plsc.VectorSubcoreMesh(core_axis_name="c", subcore_axis_name="s")
    return pl.kernel(
        _sc_body,
        out_type=[
            jax.ShapeDtypeStruct((NC, N, D), jnp.float32),
            jax.ShapeDtypeStruct((NC, N, D), jnp.float32),
        ],
        mesh=mesh,
        scratch_types=[
            pltpu.VMEM_SHARED((N, D), jnp.float32),
            pltpu.VMEM((CH,), jnp.int32),
            pltpu.VMEM((CH,), jnp.int32),
            pltpu.VMEM((CH, D), jnp.float32),
            pltpu.SemaphoreType.DMA,
        ],
    )(x, src, dst)


def _tc_body(x_ref, sp_ref, cp_ref, ws_ref, wn_ref, bn_ref,
             w1_ref, b1_ref, w2_ref, b2_ref, o_ref):
    summed = sp_ref[0] + sp_ref[1]
    deg = cp_ref[0, :, 0:1] + cp_ref[1, :, 0:1]
    agg = summed / jnp.maximum(deg, 1.0)
    h = jnp.dot(x_ref[...], ws_ref[...], preferred_element_type=jnp.float32)
    h = h + jnp.dot(agg, wn_ref[...], preferred_element_type=jnp.float32)
    h = jnp.maximum(h + bn_ref[...], 0.0)
    h = jnp.maximum(
        jnp.dot(h, w1_ref[...], preferred_element_type=jnp.float32) + b1_ref[...], 0.0)
    o_ref[...] = (
        jnp.dot(h, w2_ref[...], preferred_element_type=jnp.float32) + b2_ref[...])


@jax.jit
def _tc_mlp(x, sum_p, cnt_p, W_self, W_neigh, b_neigh, W1, b1, W2, b2):
    B = 2000
    grid = (N // B,)
    wspec = pl.BlockSpec((128, 128), lambda i: (0, 0))
    bspec = pl.BlockSpec((1, 128), lambda i: (0, 0))
    return pl.pallas_call(
        _tc_body,
        grid=grid,
        in_specs=[
            pl.BlockSpec((B, D), lambda i: (i, 0)),
            pl.BlockSpec((NC, B, D), lambda i: (0, i, 0)),
            pl.BlockSpec((NC, B, D), lambda i: (0, i, 0)),
            wspec, wspec, bspec, wspec, bspec, wspec, bspec,
        ],
        out_specs=pl.BlockSpec((B, D), lambda i: (i, 0)),
        out_shape=jax.ShapeDtypeStruct((N, D), jnp.float32),
    )(x, sum_p, cnt_p, W_self, W_neigh, b_neigh, W1, b1, W2, b2)


def kernel(x, edge_index, W_self, W_neigh, b_neigh, W1, b1, W2, b2):
    src = edge_index[0].astype(jnp.int32)
    dst = edge_index[1].astype(jnp.int32)
    sum_p, cnt_p = _sc_aggregate(x, src, dst)
    return _tc_mlp(x, sum_p, cnt_p, W_self, W_neigh,
                   b_neigh.reshape(1, D), W1, b1.reshape(1, D),
                   W2, b2.reshape(1, D))


# speedup vs baseline: 7.6036x; 1.5655x over previous
import jax
import jax.numpy as jnp
from jax import lax
from jax.experimental import pallas as pl
from jax.experimental.pallas import tpu as pltpu
from jax.experimental.pallas import tpu_sc as plsc

N = 10000
E = 320000
D = 128

NC = 2
NS = 16
NW = NC * NS
EPW = E // NW
CH = 80
NCH = EPW // CH
RPT = 624
REM = N - NS * RPT
ZR = 48


def _sc_body(x_hbm, src_hbm, dst_hbm, sum_out, cnt_out,
             acc, idx_s0, idx_s1, idx_d0, idx_d1, rows0, rows1,
             sem_g0, sem_g1, sem_s0, sem_s1):
    c = lax.axis_index("c")
    s = lax.axis_index("s")
    wid = s * NC + c
    rbase = s * RPT
    ebase = wid * EPW

    idx_s = (idx_s0, idx_s1)
    idx_d = (idx_d0, idx_d1)
    rows = (rows0, rows1)
    sem_g = (sem_g0, sem_g1)
    sem_s = (sem_s0, sem_s1)

    def _fill_rows(ref, nrows, vec):
        def _f(i, _):
            for k in range(D // 16):
                ref[i, pl.ds(16 * k, 16)] = vec
            return 0
        lax.fori_loop(0, nrows, _f, 0)

    def _zero_acc():
        for m in range(RPT // ZR):
            pltpu.sync_copy(rows0.at[pl.ds(0, ZR)],
                            acc.at[pl.ds(rbase + m * ZR, ZR)])

        @pl.when(s == NS - 1)
        def _zero_rem():
            pltpu.sync_copy(rows0.at[pl.ds(0, REM)],
                            acc.at[pl.ds(NS * RPT, REM)])

    def _writeback(out_hbm):
        nwb = RPT // ZR
        descs = [None, None]
        for m in range(nwb):
            b = m % 2
            if descs[b] is not None:
                descs[b].wait()
            pltpu.sync_copy(acc.at[pl.ds(rbase + m * ZR, ZR)],
                            rows[b].at[pl.ds(0, ZR)])
            descs[b] = pltpu.async_copy(
                rows[b].at[pl.ds(0, ZR)],
                out_hbm.at[c, pl.ds(rbase + m * ZR, ZR)], sem_g[b])
        for b in range(2):
            if descs[b] is not None:
                descs[b].wait()

        @pl.when(s == NS - 1)
        def _wb_rem():
            pltpu.sync_copy(acc.at[pl.ds(NS * RPT, REM)], rows0.at[pl.ds(0, REM)])
            pltpu.sync_copy(rows0.at[pl.ds(0, REM)],
                            out_hbm.at[c, pl.ds(NS * RPT, REM)])

    def _load_idx(j, b, also_src):
        off = pl.multiple_of(ebase + j * CH, 8)
        if also_src:
            pltpu.sync_copy(src_hbm.at[pl.ds(off, CH)], idx_s[b])
        pltpu.sync_copy(dst_hbm.at[pl.ds(off, CH)], idx_d[b])

    def _gather_start(b):
        return pltpu.async_copy(x_hbm.at[idx_s[b]], rows[b], sem_g[b])

    def _gather_drain(b):
        pltpu.make_async_copy(x_hbm.at[idx_s[b]], rows[b], sem_g[b]).wait()

    def _scat_start(b):
        return pltpu.async_copy(rows[b], acc.at[idx_d[b]], sem_s[b], add=True)

    def _scat_drain(b):
        pltpu.make_async_copy(rows[b], acc.at[idx_d[b]], sem_s[b]).wait()

    zero16 = jnp.zeros((16,), jnp.float32)
    one16 = jnp.ones((16,), jnp.float32)

    _fill_rows(rows0, ZR, zero16)
    _zero_acc()
    plsc.subcore_barrier()

    _load_idx(0, 0, True)
    _gather_start(0)
    _load_idx(1, 1, True)
    _gather_start(1)

    def _pair1(p, _):
        j = 2 * p
        for b in range(2):
            _gather_drain(b)
            _scat_start(b)
        for b in range(2):
            _scat_drain(b)
            _load_idx(j + 2 + b, b, True)
            _gather_start(b)
        return 0
    lax.fori_loop(0, (NCH - 3) // 2, _pair1, 0)

    _gather_drain(0)
    _scat_start(0)
    _gather_drain(1)
    _scat_start(1)
    _scat_drain(0)
    _load_idx(NCH - 1, 0, True)
    _gather_start(0)
    _gather_drain(0)
    _scat_start(0)
    _scat_drain(0)
    _scat_drain(1)
    plsc.subcore_barrier()

    _writeback(sum_out)
    plsc.subcore_barrier()

    _fill_rows(rows0, ZR, zero16)
    _zero_acc()
    plsc.subcore_barrier()
    _fill_rows(rows0, CH, one16)
    _fill_rows(rows1, CH, one16)

    _load_idx(0, 0, False)
    _scat_start(0)
    _load_idx(1, 1, False)
    _scat_start(1)

    def _pair2(p, _):
        j = 2 * p
        for b in range(2):
            _scat_drain(b)
            _load_idx(j + 2 + b, b, False)
            _scat_start(b)
        return 0
    lax.fori_loop(0, (NCH - 3) // 2, _pair2, 0)

    _scat_drain(0)
    _load_idx(NCH - 1, 0, False)
    _scat_start(0)
    _scat_drain(0)
    _scat_drain(1)
    plsc.subcore_barrier()

    _writeback(cnt_out)


@jax.jit
def _sc_aggregate(x, src, dst):
    mesh = plsc.VectorSubcoreMesh(core_axis_name="c", subcore_axis_name="s")
    return pl.kernel(
        _sc_body,
        out_type=[
            jax.ShapeDtypeStruct((NC, N, D), jnp.float32),
            jax.ShapeDtypeStruct((NC, N, D), jnp.float32),
        ],
        mesh=mesh,
        scratch_types=[
            pltpu.VMEM_SHARED((N, D), jnp.float32),
            pltpu.VMEM((CH,), jnp.int32),
            pltpu.VMEM((CH,), jnp.int32),
            pltpu.VMEM((CH,), jnp.int32),
            pltpu.VMEM((CH,), jnp.int32),
            pltpu.VMEM((CH, D), jnp.float32),
            pltpu.VMEM((CH, D), jnp.float32),
            pltpu.SemaphoreType.DMA,
            pltpu.SemaphoreType.DMA,
            pltpu.SemaphoreType.DMA,
            pltpu.SemaphoreType.DMA,
        ],
    )(x, src, dst)


def _tc_body(x_ref, sp_ref, cp_ref, ws_ref, wn_ref, bn_ref,
             w1_ref, b1_ref, w2_ref, b2_ref, o_ref):
    summed = sp_ref[0] + sp_ref[1]
    deg = cp_ref[0, :, 0:1] + cp_ref[1, :, 0:1]
    agg = summed / jnp.maximum(deg, 1.0)
    h = jnp.dot(x_ref[...], ws_ref[...], preferred_element_type=jnp.float32)
    h = h + jnp.dot(agg, wn_ref[...], preferred_element_type=jnp.float32)
    h = jnp.maximum(h + bn_ref[...], 0.0)
    h = jnp.maximum(
        jnp.dot(h, w1_ref[...], preferred_element_type=jnp.float32) + b1_ref[...], 0.0)
    o_ref[...] = (
        jnp.dot(h, w2_ref[...], preferred_element_type=jnp.float32) + b2_ref[...])


@jax.jit
def _tc_mlp(x, sum_p, cnt_p, W_self, W_neigh, b_neigh, W1, b1, W2, b2):
    B = 2000
    grid = (N // B,)
    wspec = pl.BlockSpec((128, 128), lambda i: (0, 0))
    bspec = pl.BlockSpec((1, 128), lambda i: (0, 0))
    return pl.pallas_call(
        _tc_body,
        grid=grid,
        in_specs=[
            pl.BlockSpec((B, D), lambda i: (i, 0)),
            pl.BlockSpec((NC, B, D), lambda i: (0, i, 0)),
            pl.BlockSpec((NC, B, D), lambda i: (0, i, 0)),
            wspec, wspec, bspec, wspec, bspec, wspec, bspec,
        ],
        out_specs=pl.BlockSpec((B, D), lambda i: (i, 0)),
        out_shape=jax.ShapeDtypeStruct((N, D), jnp.float32),
    )(x, sum_p, cnt_p, W_self, W_neigh, b_neigh, W1, b1, W2, b2)


def kernel(x, edge_index, W_self, W_neigh, b_neigh, W1, b1, W2, b2):
    src = edge_index[0].astype(jnp.int32)
    dst = edge_index[1].astype(jnp.int32)
    sum_p, cnt_p = _sc_aggregate(x, src, dst)
    return _tc_mlp(x, sum_p, cnt_p, W_self, W_neigh,
                   b_neigh.reshape(1, D), W1, b1.reshape(1, D),
                   W2, b2.reshape(1, D))


# speedup vs baseline: 8.8996x; 1.1705x over previous
import jax
import jax.numpy as jnp
from jax import lax
from jax.experimental import pallas as pl
from jax.experimental.pallas import tpu as pltpu
from jax.experimental.pallas import tpu_sc as plsc

N = 10000
E = 320000
D = 128

NC = 2
NS = 16
NW = NC * NS
EPW = E // NW
CH = 128
NCHF = E // (CH * NW)
EXTRA = (E // CH) - NCHF * NW
RPT = 624
REM = N - NS * RPT
ZR = 48


def _sc_body(x_hbm, src_hbm, dst_hbm, sum_out, cnt_out,
             acc, idx_s0, idx_s1, idx_d0, idx_d1, rows0, rows1,
             sem_g0, sem_g1, sem_s0, sem_s1):
    c = lax.axis_index("c")
    s = lax.axis_index("s")
    wid = s * NC + c
    rbase = s * RPT
    ebase = wid * EPW

    idx_s = (idx_s0, idx_s1)
    idx_d = (idx_d0, idx_d1)
    rows = (rows0, rows1)
    sem_g = (sem_g0, sem_g1)
    sem_s = (sem_s0, sem_s1)

    def _fill_rows(ref, nrows, vec):
        def _f(i, _):
            for k in range(D // 16):
                ref[i, pl.ds(16 * k, 16)] = vec
            return 0
        lax.fori_loop(0, nrows, _f, 0)

    def _zero_acc():
        for m in range(RPT // ZR):
            pltpu.sync_copy(rows0.at[pl.ds(0, ZR)],
                            acc.at[pl.ds(rbase + m * ZR, ZR)])

        @pl.when(s == NS - 1)
        def _zero_rem():
            pltpu.sync_copy(rows0.at[pl.ds(0, REM)],
                            acc.at[pl.ds(NS * RPT, REM)])

    def _writeback(out_hbm):
        nwb = RPT // ZR
        descs = [None, None]
        for m in range(nwb):
            b = m % 2
            if descs[b] is not None:
                descs[b].wait()
            pltpu.sync_copy(acc.at[pl.ds(rbase + m * ZR, ZR)],
                            rows[b].at[pl.ds(0, ZR)])
            descs[b] = pltpu.async_copy(
                rows[b].at[pl.ds(0, ZR)],
                out_hbm.at[c, pl.ds(rbase + m * ZR, ZR)], sem_g[b])
        for b in range(2):
            if descs[b] is not None:
                descs[b].wait()

        @pl.when(s == NS - 1)
        def _wb_rem():
            pltpu.sync_copy(acc.at[pl.ds(NS * RPT, REM)], rows0.at[pl.ds(0, REM)])
            pltpu.sync_copy(rows0.at[pl.ds(0, REM)],
                            out_hbm.at[c, pl.ds(NS * RPT, REM)])

    def _load_idx(i, b, also_src):
        off = pl.multiple_of((wid + NW * i) * CH, 8)
        if also_src:
            pltpu.sync_copy(src_hbm.at[pl.ds(off, CH)], idx_s[b])
        pltpu.sync_copy(dst_hbm.at[pl.ds(off, CH)], idx_d[b])

    def _gather_start(b):
        return pltpu.async_copy(x_hbm.at[idx_s[b]], rows[b], sem_g[b])

    def _gather_drain(b):
        pltpu.make_async_copy(x_hbm.at[idx_s[b]], rows[b], sem_g[b]).wait()

    def _scat_start(b):
        return pltpu.async_copy(rows[b], acc.at[idx_d[b]], sem_s[b], add=True)

    def _scat_drain(b):
        pltpu.make_async_copy(rows[b], acc.at[idx_d[b]], sem_s[b]).wait()

    zero16 = jnp.zeros((16,), jnp.float32)
    one16 = jnp.ones((16,), jnp.float32)

    _fill_rows(rows0, ZR, zero16)
    _zero_acc()
    plsc.subcore_barrier()

    _load_idx(0, 0, True)
    _gather_start(0)
    _load_idx(1, 1, True)
    _gather_start(1)

    def _pair1(p, _):
        j = 2 * p
        for b in range(2):
            _gather_drain(b)
            _scat_start(b)
        for b in range(2):
            _scat_drain(b)
            _load_idx(j + 2 + b, b, True)
            _gather_start(b)
        return 0
    lax.fori_loop(0, (NCHF - 2) // 2, _pair1, 0)

    _gather_drain(0)
    _scat_start(0)
    _gather_drain(1)
    _scat_start(1)
    _scat_drain(0)
    _scat_drain(1)

    @pl.when(wid < EXTRA)
    def _extra1():
        off = pl.multiple_of((NW * NCHF + wid) * CH, 8)
        pltpu.sync_copy(src_hbm.at[pl.ds(off, CH)], idx_s[0])
        pltpu.sync_copy(dst_hbm.at[pl.ds(off, CH)], idx_d[0])
        _gather_start(0)
        _gather_drain(0)
        _scat_start(0)
        _scat_drain(0)
    plsc.subcore_barrier()

    _writeback(sum_out)
    plsc.subcore_barrier()

    _fill_rows(rows0, ZR, zero16)
    _zero_acc()
    plsc.subcore_barrier()
    _fill_rows(rows0, CH, one16)
    _fill_rows(rows1, CH, one16)

    _load_idx(0, 0, False)
    _scat_start(0)
    _load_idx(1, 1, False)
    _scat_start(1)

    def _pair2(p, _):
        j = 2 * p
        for b in range(2):
            _scat_drain(b)
            _load_idx(j + 2 + b, b, False)
            _scat_start(b)
        return 0
    lax.fori_loop(0, (NCHF - 2) // 2, _pair2, 0)

    _scat_drain(0)
    _scat_drain(1)

    @pl.when(wid < EXTRA)
    def _extra2():
        off = pl.multiple_of((NW * NCHF + wid) * CH, 8)
        pltpu.sync_copy(dst_hbm.at[pl.ds(off, CH)], idx_d[0])
        _scat_start(0)
        _scat_drain(0)
    plsc.subcore_barrier()

    _writeback(cnt_out)


@jax.jit
def _sc_aggregate(x, src, dst):
    mesh = plsc.VectorSubcoreMesh(core_axis_name="c", subcore_axis_name="s")
    return pl.kernel(
        _sc_body,
        out_type=[
            jax.ShapeDtypeStruct((NC, N, D), jnp.float32),
            jax.ShapeDtypeStruct((NC, N, D), jnp.float32),
        ],
        mesh=mesh,
        scratch_types=[
            pltpu.VMEM_SHARED((N, D), jnp.float32),
            pltpu.VMEM((CH,), jnp.int32),
            pltpu.VMEM((CH,), jnp.int32),
            pltpu.VMEM((CH,), jnp.int32),
            pltpu.VMEM((CH,), jnp.int32),
            pltpu.VMEM((CH, D), jnp.float32),
            pltpu.VMEM((CH, D), jnp.float32),
            pltpu.SemaphoreType.DMA,
            pltpu.SemaphoreType.DMA,
            pltpu.SemaphoreType.DMA,
            pltpu.SemaphoreType.DMA,
        ],
    )(x, src, dst)


def _tc_body(x_ref, sp_ref, cp_ref, ws_ref, wn_ref, bn_ref,
             w1_ref, b1_ref, w2_ref, b2_ref, o_ref):
    summed = sp_ref[0] + sp_ref[1]
    deg = cp_ref[0, :, 0:1] + cp_ref[1, :, 0:1]
    agg = summed / jnp.maximum(deg, 1.0)
    h = jnp.dot(x_ref[...], ws_ref[...], preferred_element_type=jnp.float32)
    h = h + jnp.dot(agg, wn_ref[...], preferred_element_type=jnp.float32)
    h = jnp.maximum(h + bn_ref[...], 0.0)
    h = jnp.maximum(
        jnp.dot(h, w1_ref[...], preferred_element_type=jnp.float32) + b1_ref[...], 0.0)
    o_ref[...] = (
        jnp.dot(h, w2_ref[...], preferred_element_type=jnp.float32) + b2_ref[...])


@jax.jit
def _tc_mlp(x, sum_p, cnt_p, W_self, W_neigh, b_neigh, W1, b1, W2, b2):
    B = 2000
    grid = (N // B,)
    wspec = pl.BlockSpec((128, 128), lambda i: (0, 0))
    bspec = pl.BlockSpec((1, 128), lambda i: (0, 0))
    return pl.pallas_call(
        _tc_body,
        grid=grid,
        in_specs=[
            pl.BlockSpec((B, D), lambda i: (i, 0)),
            pl.BlockSpec((NC, B, D), lambda i: (0, i, 0)),
            pl.BlockSpec((NC, B, D), lambda i: (0, i, 0)),
            wspec, wspec, bspec, wspec, bspec, wspec, bspec,
        ],
        out_specs=pl.BlockSpec((B, D), lambda i: (i, 0)),
        out_shape=jax.ShapeDtypeStruct((N, D), jnp.float32),
    )(x, sum_p, cnt_p, W_self, W_neigh, b_neigh, W1, b1, W2, b2)


def kernel(x, edge_index, W_self, W_neigh, b_neigh, W1, b1, W2, b2):
    src = edge_index[0].astype(jnp.int32)
    dst = edge_index[1].astype(jnp.int32)
    sum_p, cnt_p = _sc_aggregate(x, src, dst)
    return _tc_mlp(x, sum_p, cnt_p, W_self, W_neigh,
                   b_neigh.reshape(1, D), W1, b1.reshape(1, D),
                   W2, b2.reshape(1, D))
